# baseline (speedup 1.0000x reference)
import jax
import jax.numpy as jnp
from jax.experimental import pallas as pl
from jax.experimental.pallas import tpu as pltpu

N = 10000
D = 128
CHUNKS = [40, 160] + [200] * 49
OFFSETS = [sum(CHUNKS[:i]) for i in range(len(CHUNKS))]
NBUF = 6
BMAX = max(CHUNKS)


def _copy_in(adj_hbm, bufs, in_sems, i, slot):
    sz = CHUNKS[i]
    return pltpu.make_async_copy(
        adj_hbm.at[pl.ds(OFFSETS[i], sz), :],
        bufs.at[slot, pl.ds(0, sz), :],
        in_sems.at[slot],
    )


def _copy_out(out_hbm, obuf, out_sems, i, oslot):
    sz = CHUNKS[i]
    return pltpu.make_async_copy(
        obuf.at[oslot, pl.ds(0, sz), :],
        out_hbm.at[pl.ds(OFFSETS[i], sz), :],
        out_sems.at[oslot],
    )


def _stream_matmul(adj_hbm, x_ref, out_hbm, bufs, obuf, in_sems, out_sems):
    n = len(CHUNKS)
    for slot in range(NBUF):
        _copy_in(adj_hbm, bufs, in_sems, slot, slot).start()
    for i in range(n):
        slot = i % NBUF
        oslot = i % 2
        _copy_in(adj_hbm, bufs, in_sems, i, slot).wait()
        if i >= 2:
            _copy_out(out_hbm, obuf, out_sems, i - 2, oslot).wait()
        sz = CHUNKS[i]
        obuf[oslot, pl.ds(0, sz), :] = jnp.dot(
            bufs[slot, pl.ds(0, sz), :], x_ref[...],
            preferred_element_type=jnp.float32,
        )
        _copy_out(out_hbm, obuf, out_sems, i, oslot).start()
        if i + NBUF < n:
            _copy_in(adj_hbm, bufs, in_sems, i + NBUF, slot).start()
    _copy_out(out_hbm, obuf, out_sems, n - 2, n % 2).wait()
    _copy_out(out_hbm, obuf, out_sems, n - 1, (n - 1) % 2).wait()


def kernel(x, adj):
    return pl.pallas_call(
        _stream_matmul,
        in_specs=[
            pl.BlockSpec(memory_space=pl.MemorySpace.ANY),
            pl.BlockSpec((N, D), lambda: (0, 0)),
        ],
        out_specs=pl.BlockSpec(memory_space=pl.MemorySpace.ANY),
        out_shape=jax.ShapeDtypeStruct((N, D), jnp.float32),
        scratch_shapes=[
            pltpu.VMEM((NBUF, BMAX, N), jnp.float32),
            pltpu.VMEM((2, BMAX, D), jnp.float32),
            pltpu.SemaphoreType.DMA((NBUF,)),
            pltpu.SemaphoreType.DMA((2,)),
        ],
    )(adj, x)


# speedup vs baseline: 1.0162x; 1.0162x over previous
import jax
import jax.numpy as jnp
from jax.experimental import pallas as pl
from jax.experimental.pallas import tpu as pltpu

N = 10000
D = 128
BM = 200


def _matmul_block(adj_ref, x_ref, out_ref):
    out_ref[...] = jnp.dot(
        adj_ref[...], x_ref[...], preferred_element_type=jnp.float32
    )


def kernel(x, adj):
    grid = (N // BM,)
    return pl.pallas_call(
        _matmul_block,
        grid=grid,
        in_specs=[
            pl.BlockSpec((BM, N), lambda i: (i, 0)),
            pl.BlockSpec((N, D), lambda i: (0, 0)),
        ],
        out_specs=pl.BlockSpec((BM, D), lambda i: (i, 0)),
        out_shape=jax.ShapeDtypeStruct((N, D), jnp.float32),
        compiler_params=pltpu.CompilerParams(
            dimension_semantics=("parallel",),
        ),
    )(adj, x)
